# baseline (speedup 1.0000x reference)
import functools

import jax
import jax.numpy as jnp
from jax import lax
from jax.experimental import pallas as pl
from jax.experimental.pallas import tpu as pltpu
from jax.experimental.pallas import tpu_sc as plsc

HIDDEN = 128
MAX_SEQ = 512
LANES = 16


def _tables_body(pe_ref, w_ref, b_ref, t1_ref, t2_ref):
    pe = pe_ref[...]
    w = w_ref[...]
    t1_ref[...] = jnp.dot(pe, w[:HIDDEN, :], preferred_element_type=jnp.float32)
    t2_ref[...] = (
        jnp.dot(pe, w[HIDDEN:, :], preferred_element_type=jnp.float32) + b_ref[...]
    )


def _precompute_tables(pe_padded, W, b):
    n = pe_padded.shape[0]
    return pl.pallas_call(
        _tables_body,
        out_shape=(
            jax.ShapeDtypeStruct((n, HIDDEN), jnp.float32),
            jax.ShapeDtypeStruct((n, HIDDEN), jnp.float32),
        ),
    )(pe_padded, W, b.reshape(1, HIDDEN))


def _idx_body(ps_ref, pv_ref, iss_ref, iee_ref):
    s = ps_ref[0, 0, :]
    e = pv_ref[0, 0, :]
    iss_ref[0] = s[:, None] - s[None, :] + MAX_SEQ
    iee_ref[0] = e[:, None] - e[None, :] + MAX_SEQ


def _precompute_indices(pos_s, pos_e):
    batch, seq = pos_s.shape
    iss, iee = pl.pallas_call(
        _idx_body,
        grid=(batch,),
        in_specs=[
            pl.BlockSpec((1, 1, seq), lambda b: (b, 0, 0)),
            pl.BlockSpec((1, 1, seq), lambda b: (b, 0, 0)),
        ],
        out_specs=[
            pl.BlockSpec((1, seq, seq), lambda b: (b, 0, 0)),
            pl.BlockSpec((1, seq, seq), lambda b: (b, 0, 0)),
        ],
        out_shape=(
            jax.ShapeDtypeStruct((batch, seq, seq), jnp.int32),
            jax.ShapeDtypeStruct((batch, seq, seq), jnp.int32),
        ),
    )(pos_s.reshape(batch, 1, seq), pos_e.reshape(batch, 1, seq))
    return iss.reshape(batch * seq, seq), iee.reshape(batch * seq, seq)


def _pack_bf16_pairs(t):
    n = t.shape[0]
    g = t.astype(jnp.bfloat16).reshape(n, 4, 2, 16)
    lo = jax.lax.bitcast_convert_type(g[:, :, 0, :], jnp.uint16).astype(jnp.uint32)
    hi = jax.lax.bitcast_convert_type(g[:, :, 1, :], jnp.uint16).astype(jnp.uint32)
    packed = lo | (hi << 16)
    return jax.lax.bitcast_convert_type(packed, jnp.int32).reshape(n, 64)


def _fuse_body(iss_hbm, iee_hbm, t1_hbm, t2_hbm, out_hbm,
               idxs, idxe,
               a1_0, a2_0, a1_1, a2_1, a1_2, a2_2, a1_3, a2_3,
               a1_4, a2_4, a1_5, a2_5, a1_6, a2_6, a1_7, a2_7,
               sem_g0, sem_g1, sem_g2, sem_g3,
               sem_g4, sem_g5, sem_g6, sem_g7,
               sem_w0, sem_w1, sem_w2, sem_w3,
               sem_w4, sem_w5, sem_w6, sem_w7):
    cid = lax.axis_index("c")
    sid = lax.axis_index("s")
    wid = sid * 2 + cid

    npairs = iss_hbm.shape[0]
    seq = iss_hbm.shape[1]
    per_w = npairs // 32
    nq = 8
    q = seq // nq
    kblocks = HIDDEN // LANES

    a1 = (a1_0, a1_1, a1_2, a1_3, a1_4, a1_5, a1_6, a1_7)
    a2 = (a2_0, a2_1, a2_2, a2_3, a2_4, a2_5, a2_6, a2_7)
    sem_g = (sem_g0, sem_g1, sem_g2, sem_g3, sem_g4, sem_g5, sem_g6, sem_g7)
    sem_w = (sem_w0, sem_w1, sem_w2, sem_w3, sem_w4, sem_w5, sem_w6, sem_w7)

    base = wid * per_w
    pltpu.sync_copy(iss_hbm.at[pl.ds(base, per_w)], idxs)
    pltpu.sync_copy(iee_hbm.at[pl.ds(base, per_w)], idxe)

    def start_gather(g, s):
        pltpu.async_copy(t1_hbm.at[idxs.at[g, pl.ds(s * q, q)]],
                         a1[s], sem_g[s])
        pltpu.async_copy(t2_hbm.at[idxe.at[g, pl.ds(s * q, q)]],
                         a2[s], sem_g[s])

    def wait_gather(s):
        pltpu.make_async_copy(t1_hbm.at[idxs.at[0, pl.ds(0, q)]],
                              a1[s], sem_g[s]).wait()
        pltpu.make_async_copy(t2_hbm.at[idxe.at[0, pl.ds(0, q)]],
                              a2[s], sem_g[s]).wait()

    def compute(s):
        buf1 = a1[s]
        buf2 = a2[s]
        dst = a1[s]
        slope = jnp.float32(0.01)

        def row_body(r, c2):
            for u in range(4):
                row = 4 * r + u
                for k in range(kblocks):
                    sl = pl.ds(k * LANES, LANES)
                    x = buf1[row, sl] + buf2[row, sl]
                    dst[row, sl] = jnp.maximum(x, x * slope)
            return c2

        lax.fori_loop(0, q // 4, row_body, 0)

    def start_writeout(g, s):
        pltpu.async_copy(a1[s], out_hbm.at[base + g, pl.ds(s * q, q)],
                         sem_w[s])

    def wait_writeout(s):
        pltpu.make_async_copy(a1[s], out_hbm.at[0, pl.ds(0, q)],
                              sem_w[s]).wait()

    def pair_body(g, carry):
        for h in range(nq):
            prev = (h - 1) % nq

            @pl.when(g > 0)
            def _():
                wait_writeout(h)

            start_gather(g, h)

            if h == 0:
                @pl.when(g > 0)
                def _():
                    wait_gather(prev)
                    compute(prev)
                    start_writeout(g - 1, prev)
            else:
                wait_gather(prev)
                compute(prev)
                start_writeout(g, prev)
        return carry

    lax.fori_loop(0, per_w, pair_body, 0)

    wait_gather(nq - 1)
    compute(nq - 1)
    start_writeout(per_w - 1, nq - 1)
    for s in range(nq):
        wait_writeout(s)


def kernel(pos_s, pos_e, pe, W, b):
    batch, seq = pos_s.shape
    n_pad = ((pe.shape[0] + 7) // 8) * 8
    pe_padded = jnp.pad(pe, ((0, n_pad - pe.shape[0]), (0, 0)))
    t1, t2 = _precompute_tables(pe_padded, W, b)
    iss, iee = _precompute_indices(
        pos_s.astype(jnp.int32), pos_e.astype(jnp.int32)
    )

    mesh = plsc.VectorSubcoreMesh(core_axis_name="c", subcore_axis_name="s")
    fuse = functools.partial(
        pl.kernel,
        mesh=mesh,
        out_type=jax.ShapeDtypeStruct((batch * seq, seq, HIDDEN), jnp.float32),
        scratch_types=(
            [
                pltpu.VMEM((batch * seq // 32, seq), jnp.int32),
                pltpu.VMEM((batch * seq // 32, seq), jnp.int32),
            ]
            + [pltpu.VMEM((seq // 8, HIDDEN), jnp.float32) for _ in range(16)]
            + [pltpu.SemaphoreType.DMA for _ in range(16)]
        ),
    )(_fuse_body)
    out = fuse(iss, iee, t1, t2)
    return out.reshape(batch, seq, seq, HIDDEN)


# speedup vs baseline: 1.1511x; 1.1511x over previous
import functools

import jax
import jax.numpy as jnp
from jax import lax
from jax.experimental import pallas as pl
from jax.experimental.pallas import tpu as pltpu
from jax.experimental.pallas import tpu_sc as plsc

HIDDEN = 128
MAX_SEQ = 512
LANES = 16


def _tables_body(pe_ref, w_ref, b_ref, t1_ref, t2_ref):
    pe = pe_ref[...]
    w = w_ref[...]
    t1_ref[...] = jnp.dot(pe, w[:HIDDEN, :], preferred_element_type=jnp.float32)
    t2_ref[...] = (
        jnp.dot(pe, w[HIDDEN:, :], preferred_element_type=jnp.float32) + b_ref[...]
    )


def _precompute_tables(pe_padded, W, b):
    n = pe_padded.shape[0]
    return pl.pallas_call(
        _tables_body,
        out_shape=(
            jax.ShapeDtypeStruct((n, HIDDEN), jnp.float32),
            jax.ShapeDtypeStruct((n, HIDDEN), jnp.float32),
        ),
    )(pe_padded, W, b.reshape(1, HIDDEN))


def _idx_body(ps_ref, pv_ref, iss_ref, iee_ref):
    s = ps_ref[0, 0, :]
    e = pv_ref[0, 0, :]
    iss_ref[0] = s[:, None] - s[None, :] + MAX_SEQ
    iee_ref[0] = e[:, None] - e[None, :] + MAX_SEQ


def _precompute_indices(pos_s, pos_e):
    batch, seq = pos_s.shape
    iss, iee = pl.pallas_call(
        _idx_body,
        grid=(batch,),
        in_specs=[
            pl.BlockSpec((1, 1, seq), lambda b: (b, 0, 0)),
            pl.BlockSpec((1, 1, seq), lambda b: (b, 0, 0)),
        ],
        out_specs=[
            pl.BlockSpec((1, seq, seq), lambda b: (b, 0, 0)),
            pl.BlockSpec((1, seq, seq), lambda b: (b, 0, 0)),
        ],
        out_shape=(
            jax.ShapeDtypeStruct((batch, seq, seq), jnp.int32),
            jax.ShapeDtypeStruct((batch, seq, seq), jnp.int32),
        ),
    )(pos_s.reshape(batch, 1, seq), pos_e.reshape(batch, 1, seq))
    return iss.reshape(batch * seq, seq), iee.reshape(batch * seq, seq)


def _pack_bf16_pairs(t):
    n = t.shape[0]
    g = t.astype(jnp.bfloat16).reshape(n, 4, 2, 16)
    lo = jax.lax.bitcast_convert_type(g[:, :, 0, :], jnp.uint16).astype(jnp.uint32)
    hi = jax.lax.bitcast_convert_type(g[:, :, 1, :], jnp.uint16).astype(jnp.uint32)
    packed = lo | (hi << 16)
    return jax.lax.bitcast_convert_type(packed, jnp.int32).reshape(n, 64)


def _fuse_body(iss_hbm, iee_hbm, t1_hbm, t2_hbm, out_hbm,
               idxs, idxe,
               a1_0, a2_0, a1_1, a2_1, a1_2, a2_2, a1_3, a2_3,
               sem_g0, sem_g1, sem_g2, sem_g3,
               sem_w0, sem_w1, sem_w2, sem_w3):
    cid = lax.axis_index("c")
    sid = lax.axis_index("s")
    wid = sid * 2 + cid

    npairs = iss_hbm.shape[0]
    seq = iss_hbm.shape[1]
    per_w = npairs // 32
    nq = 4
    q = seq // nq
    kblocks = HIDDEN // LANES

    a1 = (a1_0, a1_1, a1_2, a1_3)
    a2 = (a2_0, a2_1, a2_2, a2_3)
    sem_g = (sem_g0, sem_g1, sem_g2, sem_g3)
    sem_w = (sem_w0, sem_w1, sem_w2, sem_w3)

    base = wid * per_w
    pltpu.sync_copy(iss_hbm.at[pl.ds(base, per_w)], idxs)
    pltpu.sync_copy(iee_hbm.at[pl.ds(base, per_w)], idxe)

    def start_gather(g, s):
        pltpu.async_copy(t1_hbm.at[idxs.at[g, pl.ds(s * q, q)]],
                         a1[s], sem_g[s])
        pltpu.async_copy(t2_hbm.at[idxe.at[g, pl.ds(s * q, q)]],
                         a2[s], sem_g[s])

    def wait_gather(s):
        pltpu.make_async_copy(t1_hbm.at[idxs.at[0, pl.ds(0, q)]],
                              a1[s], sem_g[s]).wait()
        pltpu.make_async_copy(t2_hbm.at[idxe.at[0, pl.ds(0, q)]],
                              a2[s], sem_g[s]).wait()

    def compute(s):
        buf1 = a1[s]
        buf2 = a2[s]
        dst = a1[s]
        slope = jnp.float32(0.01)

        def row_body(r, c2):
            for u in range(4):
                row = 4 * r + u
                for k in range(kblocks):
                    sl = pl.ds(k * LANES, LANES)
                    x = buf1[row, sl] + buf2[row, sl]
                    dst[row, sl] = jnp.maximum(x, x * slope)
            return c2

        lax.fori_loop(0, q // 4, row_body, 0)

    def start_writeout(g, s):
        pltpu.async_copy(a1[s], out_hbm.at[base + g, pl.ds(s * q, q)],
                         sem_w[s])

    def wait_writeout(s):
        pltpu.make_async_copy(a1[s], out_hbm.at[0, pl.ds(0, q)],
                              sem_w[s]).wait()

    start_gather(0, 0)
    start_gather(0, 1)

    def pair_body(g, carry):
        for h in range(nq):
            wait_gather(h)
            compute(h)
            start_writeout(g, h)

            if h < 2:
                @pl.when(g > 0)
                def _():
                    wait_writeout(h + 2)

                start_gather(g, h + 2)
            else:
                wait_writeout(h - 2)

                @pl.when(g < per_w - 1)
                def _():
                    start_gather(g + 1, h - 2)
        return carry

    lax.fori_loop(0, per_w, pair_body, 0)

    wait_writeout(nq - 2)
    wait_writeout(nq - 1)


def kernel(pos_s, pos_e, pe, W, b):
    batch, seq = pos_s.shape
    n_pad = ((pe.shape[0] + 7) // 8) * 8
    pe_padded = jnp.pad(pe, ((0, n_pad - pe.shape[0]), (0, 0)))
    t1, t2 = _precompute_tables(pe_padded, W, b)
    iss, iee = _precompute_indices(
        pos_s.astype(jnp.int32), pos_e.astype(jnp.int32)
    )

    mesh = plsc.VectorSubcoreMesh(core_axis_name="c", subcore_axis_name="s")
    fuse = functools.partial(
        pl.kernel,
        mesh=mesh,
        out_type=jax.ShapeDtypeStruct((batch * seq, seq, HIDDEN), jnp.float32),
        scratch_types=(
            [
                pltpu.VMEM((batch * seq // 32, seq), jnp.int32),
                pltpu.VMEM((batch * seq // 32, seq), jnp.int32),
            ]
            + [pltpu.VMEM((seq // 4, HIDDEN), jnp.float32) for _ in range(8)]
            + [pltpu.SemaphoreType.DMA for _ in range(8)]
        ),
    )(_fuse_body)
    out = fuse(iss, iee, t1, t2)
    return out.reshape(batch, seq, seq, HIDDEN)
